# Initial kernel scaffold; baseline (speedup 1.0000x reference)
import jax
import jax.numpy as jnp
from jax.experimental import pallas as pl


def kernel(x, aa_table, pos_table, gamma, beta):
    raise NotImplementedError("write your pallas kernel here")


# speedup vs baseline: 5.9151x; 5.9151x over previous
import functools

import jax
import jax.numpy as jnp
from jax import lax
from jax.experimental import pallas as pl
from jax.experimental.pallas import tpu as pltpu
from jax.experimental.pallas import tpu_sc as plsc

D = 128
V = 27
P = 33
B = 16384
N = B * P
EPS = 1e-5

NC, NS, L = 2, 16, 16
NW = NC * NS
PER_W = N // NW
CHUNK = 128
ROWS_PER_W = PER_W // CHUNK
VECS_PER_IDX_ROW = CHUNK // L


def _ln_table_body(aa_ref, pos_ref, g_ref, b_ref, out_ref):
    h = aa_ref[...][:, None, :] + pos_ref[...][None, :, :]
    m = jnp.mean(h, axis=-1, keepdims=True)
    c = h - m
    var = jnp.mean(c * c, axis=-1, keepdims=True)
    out_ref[...] = c * lax.rsqrt(var + EPS) * g_ref[...] + b_ref[...]


def _build_table(aa, pos, gamma, beta):
    out = pl.pallas_call(
        _ln_table_body,
        out_shape=jax.ShapeDtypeStruct((V, P, D), jnp.float32),
    )(aa, pos, gamma.reshape(1, 1, D), beta.reshape(1, 1, D))
    return out.reshape(V * P, D)


_SC_MESH = plsc.VectorSubcoreMesh(core_axis_name="c", subcore_axis_name="s")


@functools.partial(
    pl.kernel,
    out_type=jax.ShapeDtypeStruct((N, D), jnp.float32),
    mesh=_SC_MESH,
    scratch_types=[
        pltpu.VMEM((ROWS_PER_W, CHUNK), jnp.int32),
        pltpu.VMEM((ROWS_PER_W, CHUNK), jnp.int32),
        pltpu.VMEM((CHUNK, D), jnp.float32),
        pltpu.SemaphoreType.DMA,
    ],
)
def _sc_gather(x_hbm, table_hbm, out_hbm, x_v, idx_v, rows_v, sem):
    wid = lax.axis_index("s") * NC + lax.axis_index("c")
    base_elem = wid * PER_W

    pltpu.sync_copy(x_hbm.at[wid], x_v)

    def idx_row(r, _):
        for o in range(VECS_PER_IDX_ROW):
            off = r * CHUNK + o * L
            posv = (off + lax.iota(jnp.int32, L)) % P
            idx_v[r, pl.ds(o * L, L)] = x_v[r, pl.ds(o * L, L)] * P + posv
        return 0

    lax.fori_loop(0, ROWS_PER_W, idx_row, 0)

    def chunk(c, _):
        cp = pltpu.async_copy(table_hbm.at[idx_v.at[c]], rows_v, sem)
        cp.wait()
        pltpu.sync_copy(rows_v, out_hbm.at[pl.ds(base_elem + c * CHUNK, CHUNK)])
        return 0

    lax.fori_loop(0, ROWS_PER_W, chunk, 0)


def kernel(x, aa_table, pos_table, gamma, beta):
    table = _build_table(aa_table, pos_table, gamma, beta)
    x_flat = x.astype(jnp.int32).reshape(NW, ROWS_PER_W, CHUNK)
    out = _sc_gather(x_flat, table)
    return out.reshape(B, P, D)


# speedup vs baseline: 6.2393x; 1.0548x over previous
import functools

import jax
import jax.numpy as jnp
from jax import lax
from jax.experimental import pallas as pl
from jax.experimental.pallas import tpu as pltpu
from jax.experimental.pallas import tpu_sc as plsc

D = 128
V = 27
P = 33
B = 16384
N = B * P
EPS = 1e-5

NC, NS, L = 2, 16, 16
NW = NC * NS
PER_W = N // NW
CHUNK = 128
ROWS_PER_W = PER_W // CHUNK
VECS_PER_IDX_ROW = CHUNK // L
GROUP = 3
GROUP_ROWS = GROUP * CHUNK
NG = ROWS_PER_W // GROUP


def _ln_table_body(aa_ref, pos_ref, g_ref, b_ref, out_ref):
    h = aa_ref[...][:, None, :] + pos_ref[...][None, :, :]
    m = jnp.mean(h, axis=-1, keepdims=True)
    c = h - m
    var = jnp.mean(c * c, axis=-1, keepdims=True)
    out_ref[...] = c * lax.rsqrt(var + EPS) * g_ref[...] + b_ref[...]


def _build_table(aa, pos, gamma, beta):
    out = pl.pallas_call(
        _ln_table_body,
        out_shape=jax.ShapeDtypeStruct((V, P, D), jnp.float32),
    )(aa, pos, gamma.reshape(1, 1, D), beta.reshape(1, 1, D))
    return out.reshape(V * P, D)


_SC_MESH = plsc.VectorSubcoreMesh(core_axis_name="c", subcore_axis_name="s")


@functools.partial(
    pl.kernel,
    out_type=jax.ShapeDtypeStruct((N, D), jnp.float32),
    mesh=_SC_MESH,
    scratch_types=[
        pltpu.VMEM((ROWS_PER_W, CHUNK), jnp.int32),
        pltpu.VMEM((GROUP_ROWS, D), jnp.float32),
        pltpu.VMEM((GROUP_ROWS, D), jnp.float32),
        pltpu.SemaphoreType.DMA,
        pltpu.SemaphoreType.DMA,
    ],
)
def _sc_gather(x_hbm, table_hbm, out_hbm, idx_v, buf0, buf1, sem_g, sem_s):
    wid = lax.axis_index("s") * NC + lax.axis_index("c")
    base_elem = wid * PER_W

    pltpu.sync_copy(x_hbm.at[wid], idx_v)

    def idx_row(r, _):
        for o in range(VECS_PER_IDX_ROW):
            off = r * CHUNK + o * L
            posv = (off + lax.iota(jnp.int32, L)) % P
            idx_v[r, pl.ds(o * L, L)] = idx_v[r, pl.ds(o * L, L)] * P + posv
        return 0

    lax.fori_loop(0, ROWS_PER_W, idx_row, 0)

    bufs = (buf0, buf1)

    def out_slice(g):
        return out_hbm.at[pl.ds(base_elem + g * GROUP_ROWS, GROUP_ROWS)]

    def body(g, _):
        par = g % 2
        for b in range(2):
            @pl.when(par == b)
            def _issue_and_complete():
                @pl.when(g < NG)
                def _issue():
                    @pl.when(g >= 2)
                    def _free_buf():
                        pltpu.make_async_copy(bufs[b], out_slice(0), sem_s).wait()
                    for k in range(GROUP):
                        pltpu.async_copy(
                            table_hbm.at[idx_v.at[g * GROUP + k]],
                            bufs[b].at[pl.ds(k * CHUNK, CHUNK)], sem_g)
                @pl.when(g >= 1)
                def _complete():
                    pltpu.make_async_copy(out_slice(0), bufs[1 - b], sem_g).wait()
                    pltpu.async_copy(bufs[1 - b], out_slice(g - 1), sem_s)
        return 0

    lax.fori_loop(0, NG + 1, body, 0)
    pltpu.make_async_copy(buf0, out_slice(0), sem_s).wait()
    pltpu.make_async_copy(buf1, out_slice(0), sem_s).wait()


def kernel(x, aa_table, pos_table, gamma, beta):
    table = _build_table(aa_table, pos_table, gamma, beta)
    x_flat = x.astype(jnp.int32).reshape(NW, ROWS_PER_W, CHUNK)
    out = _sc_gather(x_flat, table)
    return out.reshape(B, P, D)


# speedup vs baseline: 14.0139x; 2.2461x over previous
import functools

import jax
import jax.numpy as jnp
from jax import lax
from jax.experimental import pallas as pl
from jax.experimental.pallas import tpu as pltpu
from jax.experimental.pallas import tpu_sc as plsc

D = 128
V = 27
P = 33
B = 16384
N = B * P
EPS = 1e-5

NC, NS, L = 2, 16, 16
NW = NC * NS
PER_W = N // NW
CHUNK = 128
ROWS_PER_W = PER_W // CHUNK
VECS_PER_IDX_ROW = CHUNK // L
GROUP = 3
GROUP_ROWS = GROUP * CHUNK
NG = ROWS_PER_W // GROUP


def _ln_table_body(aa_ref, pos_ref, g_ref, b_ref, out_ref):
    h = aa_ref[...][:, None, :] + pos_ref[...][None, :, :]
    m = jnp.mean(h, axis=-1, keepdims=True)
    c = h - m
    var = jnp.mean(c * c, axis=-1, keepdims=True)
    out_ref[...] = c * lax.rsqrt(var + EPS) * g_ref[...] + b_ref[...]


def _build_table(aa, pos, gamma, beta):
    out = pl.pallas_call(
        _ln_table_body,
        out_shape=jax.ShapeDtypeStruct((V, P, D), jnp.float32),
    )(aa, pos, gamma.reshape(1, 1, D), beta.reshape(1, 1, D))
    return out.reshape(V * P, D)


_SC_MESH = plsc.VectorSubcoreMesh(core_axis_name="c", subcore_axis_name="s")


@functools.partial(
    pl.kernel,
    out_type=jax.ShapeDtypeStruct((N, D), jnp.float32),
    mesh=_SC_MESH,
    scratch_types=[
        pltpu.VMEM((ROWS_PER_W, CHUNK), jnp.int32),
        pltpu.VMEM((GROUP_ROWS, D), jnp.float32),
        pltpu.VMEM((GROUP_ROWS, D), jnp.float32),
        pltpu.SemaphoreType.DMA,
        pltpu.SemaphoreType.DMA,
    ],
)
def _sc_gather(x_hbm, table_hbm, out_hbm, idx_v, buf0, buf1, sem_g, sem_s):
    wid = lax.axis_index("s") * NC + lax.axis_index("c")
    base_elem = wid * PER_W

    pltpu.sync_copy(x_hbm.at[wid], idx_v)

    def idx_row(r, _):
        for o in range(VECS_PER_IDX_ROW):
            off = base_elem + r * CHUNK + o * L
            posv = lax.shift_right_logical(off + lax.iota(jnp.int32, L), 14)
            idx_v[r, pl.ds(o * L, L)] = idx_v[r, pl.ds(o * L, L)] * P + posv
        return 0

    lax.fori_loop(0, ROWS_PER_W, idx_row, 0)

    bufs = (buf0, buf1)

    def out_slice(g):
        return out_hbm.at[pl.ds(base_elem + g * GROUP_ROWS, GROUP_ROWS)]

    def body(g, _):
        par = g % 2
        for b in range(2):
            @pl.when(par == b)
            def _issue_and_complete():
                @pl.when(g < NG)
                def _issue():
                    @pl.when(g >= 2)
                    def _free_buf():
                        pltpu.make_async_copy(bufs[b], out_slice(0), sem_s).wait()
                    for k in range(GROUP):
                        pltpu.async_copy(
                            table_hbm.at[idx_v.at[g * GROUP + k]],
                            bufs[b].at[pl.ds(k * CHUNK, CHUNK)], sem_g)
                @pl.when(g >= 1)
                def _complete():
                    pltpu.make_async_copy(out_slice(0), bufs[1 - b], sem_g).wait()
                    pltpu.async_copy(bufs[1 - b], out_slice(g - 1), sem_s)
        return 0

    lax.fori_loop(0, NG + 1, body, 0)
    pltpu.make_async_copy(buf0, out_slice(0), sem_s).wait()
    pltpu.make_async_copy(buf1, out_slice(0), sem_s).wait()


def kernel(x, aa_table, pos_table, gamma, beta):
    table = _build_table(aa_table, pos_table, gamma, beta)
    x_t = x.astype(jnp.int32).T.reshape(NW, ROWS_PER_W, CHUNK)
    out = _sc_gather(x_t, table)
    return out.reshape(P, B, D).transpose(1, 0, 2)


# speedup vs baseline: 38.8128x; 2.7696x over previous
import functools

import jax
import jax.numpy as jnp
from jax import lax
from jax.experimental import pallas as pl
from jax.experimental.pallas import tpu as pltpu
from jax.experimental.pallas import tpu_sc as plsc

D = 128
V = 27
P = 33
B = 16384
N = B * P
EPS = 1e-5

NC, NS, L = 2, 16, 16
NW = NC * NS
PER_W = N // NW
CHUNK = 128
ROWS_PER_W = PER_W // CHUNK
VECS_PER_IDX_ROW = CHUNK // L
GROUP = 3
GROUP_ROWS = GROUP * CHUNK
NG = ROWS_PER_W // GROUP


def _ln_table_body(aa_ref, pos_ref, g_ref, b_ref, out_ref):
    h = aa_ref[...][:, None, :] + pos_ref[...][None, :, :]
    m = jnp.mean(h, axis=-1, keepdims=True)
    c = h - m
    var = jnp.mean(c * c, axis=-1, keepdims=True)
    out_ref[...] = c * lax.rsqrt(var + EPS) * g_ref[...] + b_ref[...]


def _build_table(aa, pos, gamma, beta):
    out = pl.pallas_call(
        _ln_table_body,
        out_shape=jax.ShapeDtypeStruct((V, P, D), jnp.float32),
    )(aa, pos, gamma.reshape(1, 1, D), beta.reshape(1, 1, D))
    return out.reshape(V * P, D)


_SC_MESH = plsc.VectorSubcoreMesh(core_axis_name="c", subcore_axis_name="s")


@functools.partial(
    pl.kernel,
    out_type=jax.ShapeDtypeStruct((N, D), jnp.float32),
    mesh=_SC_MESH,
    scratch_types=[
        pltpu.VMEM((ROWS_PER_W, CHUNK), jnp.int32),
        pltpu.VMEM((GROUP_ROWS, D), jnp.float32),
        pltpu.VMEM((GROUP_ROWS, D), jnp.float32),
        pltpu.VMEM_SHARED((V * P, D), jnp.float32),
        pltpu.SemaphoreType.DMA,
        pltpu.SemaphoreType.DMA,
    ],
)
def _sc_gather(x_hbm, table_hbm, out_hbm, idx_v, buf0, buf1, table_sp, sem_g, sem_s):
    wid = lax.axis_index("s") * NC + lax.axis_index("c")
    base_elem = wid * PER_W

    @pl.when(lax.axis_index("s") == 0)
    def _stage_table():
        pltpu.sync_copy(table_hbm, table_sp)

    plsc.subcore_barrier()

    pltpu.sync_copy(x_hbm.at[wid], idx_v)

    def idx_row(r, _):
        for o in range(VECS_PER_IDX_ROW):
            off = base_elem + r * CHUNK + o * L
            posv = lax.shift_right_logical(off + lax.iota(jnp.int32, L), 14)
            idx_v[r, pl.ds(o * L, L)] = idx_v[r, pl.ds(o * L, L)] * P + posv
        return 0

    lax.fori_loop(0, ROWS_PER_W, idx_row, 0)

    bufs = (buf0, buf1)

    def out_slice(g):
        return out_hbm.at[pl.ds(base_elem + g * GROUP_ROWS, GROUP_ROWS)]

    def body(g, _):
        par = g % 2
        for b in range(2):
            @pl.when(par == b)
            def _issue_and_complete():
                @pl.when(g < NG)
                def _issue():
                    @pl.when(g >= 2)
                    def _free_buf():
                        pltpu.make_async_copy(bufs[b], out_slice(0), sem_s).wait()
                    for k in range(GROUP):
                        pltpu.async_copy(
                            table_sp.at[idx_v.at[g * GROUP + k]],
                            bufs[b].at[pl.ds(k * CHUNK, CHUNK)], sem_g)
                @pl.when(g >= 1)
                def _complete():
                    pltpu.make_async_copy(out_slice(0), bufs[1 - b], sem_g).wait()
                    pltpu.async_copy(bufs[1 - b], out_slice(g - 1), sem_s)
        return 0

    lax.fori_loop(0, NG + 1, body, 0)
    pltpu.make_async_copy(buf0, out_slice(0), sem_s).wait()
    pltpu.make_async_copy(buf1, out_slice(0), sem_s).wait()


def kernel(x, aa_table, pos_table, gamma, beta):
    table = _build_table(aa_table, pos_table, gamma, beta)
    x_t = x.astype(jnp.int32).T.reshape(NW, ROWS_PER_W, CHUNK)
    out = _sc_gather(x_t, table)
    return out.reshape(P, B, D).transpose(1, 0, 2)


# speedup vs baseline: 39.2905x; 1.0123x over previous
import functools

import jax
import jax.numpy as jnp
from jax import lax
from jax.experimental import pallas as pl
from jax.experimental.pallas import tpu as pltpu
from jax.experimental.pallas import tpu_sc as plsc

D = 128
V = 27
P = 33
B = 16384
N = B * P
EPS = 1e-5

NC, NS, L = 2, 16, 16
NW = NC * NS
PER_W = N // NW
CHUNK = 128
ROWS_PER_W = PER_W // CHUNK
VECS_PER_IDX_ROW = CHUNK // L
GROUP = 3
GROUP_ROWS = GROUP * CHUNK
NG = ROWS_PER_W // GROUP


def _ln_table_body(aa_ref, pos_ref, g_ref, b_ref, out_ref):
    h = aa_ref[...][:, None, :] + pos_ref[...][None, :, :]
    m = jnp.mean(h, axis=-1, keepdims=True)
    c = h - m
    var = jnp.mean(c * c, axis=-1, keepdims=True)
    out_ref[...] = c * lax.rsqrt(var + EPS) * g_ref[...] + b_ref[...]


def _build_table(aa, pos, gamma, beta):
    out = pl.pallas_call(
        _ln_table_body,
        out_shape=jax.ShapeDtypeStruct((V, P, D), jnp.float32),
    )(aa, pos, gamma.reshape(1, 1, D), beta.reshape(1, 1, D))
    return out.reshape(V * P, D)


_SC_MESH = plsc.VectorSubcoreMesh(core_axis_name="c", subcore_axis_name="s")


@functools.partial(
    pl.kernel,
    out_type=jax.ShapeDtypeStruct((N, D), jnp.float32),
    mesh=_SC_MESH,
    scratch_types=[
        pltpu.VMEM((ROWS_PER_W, CHUNK), jnp.int32),
        pltpu.VMEM((GROUP, CHUNK), jnp.int32),
        pltpu.VMEM((GROUP, CHUNK), jnp.int32),
        pltpu.VMEM((GROUP_ROWS, D), jnp.float32),
        pltpu.VMEM((GROUP_ROWS, D), jnp.float32),
        pltpu.VMEM_SHARED((V * P, D), jnp.float32),
        pltpu.SemaphoreType.DMA,
        pltpu.SemaphoreType.DMA,
    ],
)
def _sc_gather(x_hbm, table_hbm, out_hbm, x_v, idx0, idx1, buf0, buf1,
               table_sp, sem_g, sem_s):
    wid = lax.axis_index("s") * NC + lax.axis_index("c")
    base_elem = wid * PER_W

    @pl.when(lax.axis_index("s") == 0)
    def _stage_table():
        pltpu.sync_copy(table_hbm, table_sp)

    pltpu.sync_copy(x_hbm.at[wid], x_v)
    plsc.subcore_barrier()

    idxs = (idx0, idx1)
    bufs = (buf0, buf1)

    def out_slice(g):
        return out_hbm.at[pl.ds(base_elem + g * GROUP_ROWS, GROUP_ROWS)]

    def body(g, _):
        par = g % 2
        for b in range(2):
            @pl.when(par == b)
            def _issue_and_complete():
                @pl.when(g < NG)
                def _issue():
                    @pl.when(g >= 2)
                    def _free_buf():
                        pltpu.make_async_copy(bufs[b], out_slice(0), sem_s).wait()
                    for k in range(GROUP):
                        for o in range(VECS_PER_IDX_ROW):
                            off = base_elem + (g * GROUP + k) * CHUNK + o * L
                            posv = lax.shift_right_logical(
                                off + lax.iota(jnp.int32, L), 14)
                            idxs[b][k, pl.ds(o * L, L)] = (
                                x_v[g * GROUP + k, pl.ds(o * L, L)] * P + posv)
                    for k in range(GROUP):
                        pltpu.async_copy(
                            table_sp.at[idxs[b].at[k]],
                            bufs[b].at[pl.ds(k * CHUNK, CHUNK)], sem_g)
                @pl.when(g >= 1)
                def _complete():
                    pltpu.make_async_copy(out_slice(0), bufs[1 - b], sem_g).wait()
                    pltpu.async_copy(bufs[1 - b], out_slice(g - 1), sem_s)
        return 0

    lax.fori_loop(0, NG + 1, body, 0)
    pltpu.make_async_copy(buf0, out_slice(0), sem_s).wait()
    pltpu.make_async_copy(buf1, out_slice(0), sem_s).wait()


def kernel(x, aa_table, pos_table, gamma, beta):
    table = _build_table(aa_table, pos_table, gamma, beta)
    x_t = x.astype(jnp.int32).T.reshape(NW, ROWS_PER_W, CHUNK)
    out = _sc_gather(x_t, table)
    return out.reshape(P, B, D).transpose(1, 0, 2)


# speedup vs baseline: 40.1438x; 1.0217x over previous
import functools

import jax
import jax.numpy as jnp
from jax import lax
from jax.experimental import pallas as pl
from jax.experimental.pallas import tpu as pltpu
from jax.experimental.pallas import tpu_sc as plsc

D = 128
V = 27
P = 33
B = 16384
N = B * P
EPS = 1e-5

NC, NS, L = 2, 16, 16
NW = NC * NS
COLS_W = B // NW
CHUNK = 128
GROUP = 2
GROUP_ROWS = GROUP * CHUNK
GPP = COLS_W // GROUP_ROWS
NG = P * GPP
NBUF = 3
VECS = CHUNK // L


def _ln_table_body(aa_ref, pos_ref, g_ref, b_ref, out_ref):
    h = aa_ref[...][:, None, :] + pos_ref[...][None, :, :]
    m = jnp.mean(h, axis=-1, keepdims=True)
    c = h - m
    var = jnp.mean(c * c, axis=-1, keepdims=True)
    out_ref[...] = c * lax.rsqrt(var + EPS) * g_ref[...] + b_ref[...]


def _build_table(aa, pos, gamma, beta):
    out = pl.pallas_call(
        _ln_table_body,
        out_shape=jax.ShapeDtypeStruct((V, P, D), jnp.float32),
    )(aa, pos, gamma.reshape(1, 1, D), beta.reshape(1, 1, D))
    return out.reshape(V * P, D)


_SC_MESH = plsc.VectorSubcoreMesh(core_axis_name="c", subcore_axis_name="s")


@functools.partial(
    pl.kernel,
    out_type=jax.ShapeDtypeStruct((N, D), jnp.float32),
    mesh=_SC_MESH,
    scratch_types=[
        pltpu.VMEM((P, COLS_W), jnp.int32),
        pltpu.VMEM((NBUF, GROUP, CHUNK), jnp.int32),
        pltpu.VMEM((GROUP_ROWS, D), jnp.float32),
        pltpu.VMEM((GROUP_ROWS, D), jnp.float32),
        pltpu.VMEM((GROUP_ROWS, D), jnp.float32),
        pltpu.VMEM_SHARED((V * P, D), jnp.float32),
        pltpu.SemaphoreType.DMA,
        pltpu.SemaphoreType.DMA,
    ],
)
def _sc_gather(x_hbm, table_hbm, out_hbm, x_v, idx_v, buf0, buf1, buf2,
               table_sp, sem_g, sem_s):
    wid = lax.axis_index("s") * NC + lax.axis_index("c")
    col0 = wid * COLS_W

    @pl.when(lax.axis_index("s") == 0)
    def _stage_table():
        pltpu.sync_copy(table_hbm, table_sp)

    pltpu.sync_copy(x_hbm.at[:, pl.ds(col0, COLS_W)], x_v)
    plsc.subcore_barrier()

    bufs = (buf0, buf1, buf2)

    def out_slice(g):
        start = (g // GPP) * B + col0 + (g % GPP) * GROUP_ROWS
        return out_hbm.at[pl.ds(start, GROUP_ROWS)]

    def body(g, _):
        par = g % NBUF
        for b in range(NBUF):
            @pl.when(par == b)
            def _issue():
                @pl.when(g < NG)
                def _fire():
                    @pl.when(g >= NBUF)
                    def _free_buf():
                        pltpu.make_async_copy(bufs[b], out_slice(0), sem_s).wait()
                    p = g // GPP
                    cbase = (g % GPP) * GROUP_ROWS
                    for k in range(GROUP):
                        for o in range(VECS):
                            sl = pl.ds(cbase + k * CHUNK + o * L, L)
                            idx_v[b, k, pl.ds(o * L, L)] = x_v[p, sl] * P + p
                    for k in range(GROUP):
                        pltpu.async_copy(
                            table_sp.at[idx_v.at[b, k]],
                            bufs[b].at[pl.ds(k * CHUNK, CHUNK)], sem_g)
            prev = (b - 1) % NBUF
            @pl.when(jnp.logical_and(par == b, g >= 1))
            def _complete():
                pltpu.make_async_copy(out_slice(0), bufs[prev], sem_g).wait()
                pltpu.async_copy(bufs[prev], out_slice(g - 1), sem_s)
        return 0

    lax.fori_loop(0, NG + 1, body, 0)
    for b in range(NBUF):
        pltpu.make_async_copy(bufs[b], out_slice(0), sem_s).wait()


def kernel(x, aa_table, pos_table, gamma, beta):
    table = _build_table(aa_table, pos_table, gamma, beta)
    out = _sc_gather(x.astype(jnp.int32).T, table)
    return out.reshape(P, B, D).transpose(1, 0, 2)
